# baseline (speedup 1.0000x reference)
import functools

import jax
import jax.numpy as jnp
from jax import lax
from jax.experimental import pallas as pl
from jax.experimental.pallas import tpu as pltpu
from jax.experimental.pallas import tpu_sc as plsc

NC = 2
NS = 16
NW = NC * NS
LANES = 16


def _fill_zero_rows(ref, nrows, ncols):
    @pl.loop(0, nrows)
    def _(i):
        for c in range(ncols // LANES):
            ref[i, pl.ds(c * LANES, LANES)] = jnp.zeros((LANES,), jnp.float32)


def _sc_degree(dst3, n_pad):
    nwork, nchunk, k = dst3.shape
    per_tile = n_pad // NS
    mesh = plsc.VectorSubcoreMesh(core_axis_name="c", subcore_axis_name="s")

    @functools.partial(
        pl.kernel,
        out_type=jax.ShapeDtypeStruct((NC, n_pad), jnp.float32),
        mesh=mesh,
        scratch_types=[
            pltpu.VMEM((k,), jnp.float32),
            pltpu.VMEM((nchunk, k), jnp.int32),
            pltpu.VMEM((per_tile,), jnp.float32),
            pltpu.VMEM_SHARED((n_pad,), jnp.float32),
        ],
    )
    def deg_kernel(dst_hbm, out_hbm, ones_v, didx_v, zero_v, acc_sh):
        cid = lax.axis_index("c")
        sid = lax.axis_index("s")
        wid = sid * NC + cid

        for c in range(k // LANES):
            ones_v[pl.ds(c * LANES, LANES)] = jnp.ones((LANES,), jnp.float32)

        @pl.loop(0, per_tile // LANES)
        def _(i):
            zero_v[pl.ds(i * LANES, LANES)] = jnp.zeros((LANES,), jnp.float32)

        pltpu.sync_copy(zero_v, acc_sh.at[pl.ds(sid * per_tile, per_tile)])
        pltpu.sync_copy(dst_hbm.at[wid], didx_v)
        plsc.subcore_barrier()

        @pl.loop(0, nchunk)
        def _(j):
            pltpu.sync_copy(ones_v, acc_sh.at[didx_v.at[j]], add=True)

        plsc.subcore_barrier()
        pltpu.sync_copy(
            acc_sh.at[pl.ds(sid * per_tile, per_tile)],
            out_hbm.at[cid, pl.ds(sid * per_tile, per_tile)],
        )

    return deg_kernel(dst3)


def _sc_scatter(g, src3, dst3, n_pad):
    n, d = g.shape
    nwork, nchunk, k = src3.shape
    per_tile = n_pad // NS
    zr = 128
    mesh = plsc.VectorSubcoreMesh(core_axis_name="c", subcore_axis_name="s")

    @functools.partial(
        pl.kernel,
        out_type=jax.ShapeDtypeStruct((NC, n_pad, d), jnp.float32),
        mesh=mesh,
        scratch_types=[
            pltpu.VMEM((2, k), jnp.int32),
            pltpu.VMEM((2, k), jnp.int32),
            pltpu.VMEM((k, d), jnp.float32),
            pltpu.VMEM((k, d), jnp.float32),
            pltpu.VMEM_SHARED((n_pad, d), jnp.float32),
            pltpu.SemaphoreType.DMA,
            pltpu.SemaphoreType.DMA,
            pltpu.SemaphoreType.DMA,
            pltpu.SemaphoreType.DMA,
        ],
    )
    def scat_kernel(g_hbm, src_hbm, dst_hbm, out_hbm,
                    sidx_v, didx_v, rows0_v, rows1_v, acc_sh,
                    gsem0, gsem1, isem0, isem1):
        cid = lax.axis_index("c")
        sid = lax.axis_index("s")
        wid = sid * NC + cid
        rows = (rows0_v, rows1_v)
        gsem = (gsem0, gsem1)
        isem = (isem0, isem1)

        def load_idx(c, b):
            pltpu.async_copy(src_hbm.at[wid, c], sidx_v.at[b], isem[b])
            pltpu.async_copy(dst_hbm.at[wid, c], didx_v.at[b], isem[b])

        def wait_idx(c, b):
            pltpu.make_async_copy(
                src_hbm.at[wid, c], sidx_v.at[b], isem[b]).wait()
            pltpu.make_async_copy(
                dst_hbm.at[wid, c], didx_v.at[b], isem[b]).wait()

        def start_gather(b):
            pltpu.async_copy(g_hbm.at[sidx_v.at[b]], rows[b], gsem[b])

        def wait_gather(b):
            pltpu.make_async_copy(
                g_hbm.at[sidx_v.at[b]], rows[b], gsem[b]).wait()

        def scatter(b):
            pltpu.sync_copy(rows[b], acc_sh.at[didx_v.at[b]], add=True)

        _fill_zero_rows(rows0_v, k, d)

        @pl.loop(0, per_tile // k)
        def _(b):
            pltpu.sync_copy(
                rows0_v, acc_sh.at[pl.ds(sid * per_tile + b * k, k)])

        plsc.subcore_barrier()

        load_idx(0, 0)
        load_idx(1, 1)
        wait_idx(0, 0)
        start_gather(0)

        @pl.loop(0, nchunk // 2)
        def _(t):
            c0 = 2 * t
            c1 = c0 + 1
            wait_gather(0)
            wait_idx(c1, 1)
            start_gather(1)
            scatter(0)
            load_idx(jnp.minimum(c0 + 2, nchunk - 1), 0)
            wait_gather(1)
            wait_idx(jnp.minimum(c0 + 2, nchunk - 1), 0)
            start_gather(0)
            scatter(1)
            load_idx(jnp.minimum(c1 + 2, nchunk - 1), 1)

        wait_gather(0)
        wait_idx(nchunk - 1, 1)

        plsc.subcore_barrier()
        pltpu.sync_copy(
            acc_sh.at[pl.ds(sid * per_tile, per_tile)],
            out_hbm.at[cid, pl.ds(sid * per_tile, per_tile)],
        )

    return scat_kernel(g, src3, dst3)


_RB = 2000


def _dot(a, w):
    return lax.dot_general(a, w, (((1,), (0,)), ((), ())),
                           precision=lax.Precision.HIGHEST,
                           preferred_element_type=jnp.float32)


def _tc_pre(dega, degb, x, w1):
    n, d = x.shape

    def body(da_ref, db_ref, x_ref, w_ref, dis_ref, g_ref):
        deg = da_ref[...] + db_ref[...] + 1.0
        dis = lax.rsqrt(deg)
        dis_ref[...] = dis
        g_ref[...] = dis * _dot(x_ref[...], w_ref[...])

    return pl.pallas_call(
        body,
        grid=(n // _RB,),
        in_specs=[
            pl.BlockSpec((_RB, 1), lambda i: (i, 0)),
            pl.BlockSpec((_RB, 1), lambda i: (i, 0)),
            pl.BlockSpec((_RB, d), lambda i: (i, 0)),
            pl.BlockSpec((d, d), lambda i: (0, 0)),
        ],
        out_specs=[
            pl.BlockSpec((_RB, 1), lambda i: (i, 0)),
            pl.BlockSpec((_RB, d), lambda i: (i, 0)),
        ],
        out_shape=[
            jax.ShapeDtypeStruct((n, 1), jnp.float32),
            jax.ShapeDtypeStruct((n, d), jnp.float32),
        ],
    )(dega, degb, x, w1)


def _tc_layer(sa, sb, g_prev, dis, b, w_next):
    n, d = g_prev.shape

    def body(sa_ref, sb_ref, g_ref, dis_ref, b_ref, w_ref, out_ref):
        dis = dis_ref[...]
        t = dis * (sa_ref[...] + sb_ref[...] + g_ref[...]) + b_ref[...]
        h = jnp.maximum(t, 0.0)
        out_ref[...] = dis * _dot(h, w_ref[...])

    return pl.pallas_call(
        body,
        grid=(n // _RB,),
        in_specs=[
            pl.BlockSpec((_RB, d), lambda i: (i, 0)),
            pl.BlockSpec((_RB, d), lambda i: (i, 0)),
            pl.BlockSpec((_RB, d), lambda i: (i, 0)),
            pl.BlockSpec((_RB, 1), lambda i: (i, 0)),
            pl.BlockSpec((1, d), lambda i: (0, 0)),
            pl.BlockSpec((d, d), lambda i: (0, 0)),
        ],
        out_specs=pl.BlockSpec((_RB, d), lambda i: (i, 0)),
        out_shape=jax.ShapeDtypeStruct((n, d), jnp.float32),
    )(sa, sb, g_prev, dis, b, w_next)


def _tc_final(sa, sb, g_prev, dis, b):
    n, d = g_prev.shape

    def body(sa_ref, sb_ref, g_ref, dis_ref, b_ref, out_ref):
        out_ref[...] = (dis_ref[...] * (sa_ref[...] + sb_ref[...] + g_ref[...])
                        + b_ref[...])

    return pl.pallas_call(
        body,
        grid=(n // _RB,),
        in_specs=[
            pl.BlockSpec((_RB, d), lambda i: (i, 0)),
            pl.BlockSpec((_RB, d), lambda i: (i, 0)),
            pl.BlockSpec((_RB, d), lambda i: (i, 0)),
            pl.BlockSpec((_RB, 1), lambda i: (i, 0)),
            pl.BlockSpec((1, d), lambda i: (0, 0)),
        ],
        out_specs=pl.BlockSpec((_RB, d), lambda i: (i, 0)),
        out_shape=jax.ShapeDtypeStruct((n, d), jnp.float32),
    )(sa, sb, g_prev, dis, b)


def kernel(x, edge_index, W1, b1, W2, b2, W3, b3):
    n, d = x.shape
    e = edge_index.shape[1]
    k = 40
    nchunk = e // (NW * k)
    assert nchunk * NW * k == e and nchunk % 2 == 0

    src3 = edge_index[0].reshape(NW, nchunk, k)
    dst3 = edge_index[1].reshape(NW, nchunk, k)

    k_deg = 80
    dst3_deg = edge_index[1].reshape(NW, e // (NW * k_deg), k_deg)

    n_pad = 10240 if n == 10000 else ((n + 8 * NS - 1) // (8 * NS)) * (8 * NS)
    deg = _sc_degree(dst3_deg, n_pad)
    dega = deg[0, :n].reshape(n, 1)
    degb = deg[1, :n].reshape(n, 1)

    b1r = b1.reshape(1, d)
    b2r = b2.reshape(1, d)
    b3r = b3.reshape(1, d)

    dis, g1 = _tc_pre(dega, degb, x, W1)
    s1 = _sc_scatter(g1, src3, dst3, n_pad)
    g2 = _tc_layer(s1[0], s1[1], g1, dis, b1r, W2)
    s2 = _sc_scatter(g2, src3, dst3, n_pad)
    g3 = _tc_layer(s2[0], s2[1], g2, dis, b2r, W3)
    s3 = _sc_scatter(g3, src3, dst3, n_pad)
    return _tc_final(s3[0], s3[1], g3, dis, b3r)


# speedup vs baseline: 1.4028x; 1.4028x over previous
import functools

import jax
import jax.numpy as jnp
from jax import lax
from jax.experimental import pallas as pl
from jax.experimental.pallas import tpu as pltpu
from jax.experimental.pallas import tpu_sc as plsc

NC = 2
NS = 16
NW = NC * NS
LANES = 16


def _fill_zero_rows(ref, nrows, ncols):
    @pl.loop(0, nrows)
    def _(i):
        for c in range(ncols // LANES):
            ref[i, pl.ds(c * LANES, LANES)] = jnp.zeros((LANES,), jnp.float32)


def _sc_degree(dst3, n_pad):
    nwork, nchunk, k = dst3.shape
    per_tile = n_pad // NS
    mesh = plsc.VectorSubcoreMesh(core_axis_name="c", subcore_axis_name="s")

    @functools.partial(
        pl.kernel,
        out_type=jax.ShapeDtypeStruct((NC, n_pad), jnp.float32),
        mesh=mesh,
        scratch_types=[
            pltpu.VMEM((k,), jnp.float32),
            pltpu.VMEM((nchunk, k), jnp.int32),
            pltpu.VMEM((per_tile,), jnp.float32),
            pltpu.VMEM_SHARED((n_pad,), jnp.float32),
        ],
    )
    def deg_kernel(dst_hbm, out_hbm, ones_v, didx_v, zero_v, acc_sh):
        cid = lax.axis_index("c")
        sid = lax.axis_index("s")
        wid = sid * NC + cid

        for c in range(k // LANES):
            ones_v[pl.ds(c * LANES, LANES)] = jnp.ones((LANES,), jnp.float32)

        @pl.loop(0, per_tile // LANES)
        def _(i):
            zero_v[pl.ds(i * LANES, LANES)] = jnp.zeros((LANES,), jnp.float32)

        pltpu.sync_copy(zero_v, acc_sh.at[pl.ds(sid * per_tile, per_tile)])
        pltpu.sync_copy(dst_hbm.at[wid], didx_v)
        plsc.subcore_barrier()

        @pl.loop(0, nchunk)
        def _(j):
            pltpu.sync_copy(ones_v, acc_sh.at[didx_v.at[j]], add=True)

        plsc.subcore_barrier()
        pltpu.sync_copy(
            acc_sh.at[pl.ds(sid * per_tile, per_tile)],
            out_hbm.at[cid, pl.ds(sid * per_tile, per_tile)],
        )

    return deg_kernel(dst3)


def _sc_scatter(g, src_flat, dst_flat, k, n_pad):
    n, d = g.shape
    e = src_flat.shape[0]
    nchunk = e // (NW * k)
    per_tile = n_pad // NS
    zr = 128
    mesh = plsc.VectorSubcoreMesh(core_axis_name="c", subcore_axis_name="s")

    @functools.partial(
        pl.kernel,
        out_type=jax.ShapeDtypeStruct((NC, n_pad, d), jnp.float32),
        mesh=mesh,
        scratch_types=[
            pltpu.VMEM((2, k), jnp.int32),
            pltpu.VMEM((2, k), jnp.int32),
            pltpu.VMEM((k, d), jnp.float32),
            pltpu.VMEM((k, d), jnp.float32),
            pltpu.VMEM_SHARED((n_pad, d), jnp.float32),
            pltpu.SemaphoreType.DMA,
            pltpu.SemaphoreType.DMA,
            pltpu.SemaphoreType.DMA,
            pltpu.SemaphoreType.DMA,
        ],
    )
    def scat_kernel(g_hbm, src_hbm, dst_hbm, out_hbm,
                    sidx_v, didx_v, rows0_v, rows1_v, acc_sh,
                    gsem0, gsem1, isem0, isem1):
        cid = lax.axis_index("c")
        sid = lax.axis_index("s")
        wid = sid * NC + cid
        base = wid * (nchunk * k)
        rows = (rows0_v, rows1_v)
        gsem = (gsem0, gsem1)
        isem = (isem0, isem1)

        def load_idx(c, b):
            pltpu.async_copy(
                src_hbm.at[pl.ds(base + c * k, k)], sidx_v.at[b], isem[b])
            pltpu.async_copy(
                dst_hbm.at[pl.ds(base + c * k, k)], didx_v.at[b], isem[b])

        def wait_idx(c, b):
            pltpu.make_async_copy(
                src_hbm.at[pl.ds(base + c * k, k)], sidx_v.at[b],
                isem[b]).wait()
            pltpu.make_async_copy(
                dst_hbm.at[pl.ds(base + c * k, k)], didx_v.at[b],
                isem[b]).wait()

        def start_gather(b):
            pltpu.async_copy(g_hbm.at[sidx_v.at[b]], rows[b], gsem[b])

        def wait_gather(b):
            pltpu.make_async_copy(
                g_hbm.at[sidx_v.at[b]], rows[b], gsem[b]).wait()

        def scatter(b):
            pltpu.sync_copy(rows[b], acc_sh.at[didx_v.at[b]], add=True)

        _fill_zero_rows(rows0_v, k, d)

        @pl.loop(0, per_tile // k)
        def _(b):
            pltpu.sync_copy(
                rows0_v, acc_sh.at[pl.ds(sid * per_tile + b * k, k)])

        plsc.subcore_barrier()

        load_idx(0, 0)
        load_idx(1, 1)
        wait_idx(0, 0)
        start_gather(0)

        @pl.loop(0, nchunk // 2)
        def _(t):
            c0 = 2 * t
            c1 = c0 + 1
            wait_gather(0)
            wait_idx(c1, 1)
            start_gather(1)
            scatter(0)
            load_idx(jnp.minimum(c0 + 2, nchunk - 1), 0)
            wait_gather(1)
            wait_idx(jnp.minimum(c0 + 2, nchunk - 1), 0)
            start_gather(0)
            scatter(1)
            load_idx(jnp.minimum(c1 + 2, nchunk - 1), 1)

        wait_gather(0)
        if nchunk % 2 == 1:
            scatter(0)
        wait_idx(nchunk - 1, 1)

        plsc.subcore_barrier()
        pltpu.sync_copy(
            acc_sh.at[pl.ds(sid * per_tile, per_tile)],
            out_hbm.at[cid, pl.ds(sid * per_tile, per_tile)],
        )

    return scat_kernel(g, src_flat, dst_flat)


_RB = 2000


def _dot(a, w):
    return lax.dot_general(a, w, (((1,), (0,)), ((), ())),
                           precision=lax.Precision.HIGHEST,
                           preferred_element_type=jnp.float32)


def _tc_pre(dega, degb, x, w1):
    n, d = x.shape

    def body(da_ref, db_ref, x_ref, w_ref, dis_ref, g_ref):
        deg = da_ref[...] + db_ref[...] + 1.0
        dis = lax.rsqrt(deg)
        dis_ref[...] = dis
        g_ref[...] = dis * _dot(x_ref[...], w_ref[...])

    return pl.pallas_call(
        body,
        grid=(n // _RB,),
        in_specs=[
            pl.BlockSpec((_RB, 1), lambda i: (i, 0)),
            pl.BlockSpec((_RB, 1), lambda i: (i, 0)),
            pl.BlockSpec((_RB, d), lambda i: (i, 0)),
            pl.BlockSpec((d, d), lambda i: (0, 0)),
        ],
        out_specs=[
            pl.BlockSpec((_RB, 1), lambda i: (i, 0)),
            pl.BlockSpec((_RB, d), lambda i: (i, 0)),
        ],
        out_shape=[
            jax.ShapeDtypeStruct((n, 1), jnp.float32),
            jax.ShapeDtypeStruct((n, d), jnp.float32),
        ],
    )(dega, degb, x, w1)


def _tc_layer(sa, sb, g_prev, dis, b, w_next):
    n, d = g_prev.shape

    def body(sa_ref, sb_ref, g_ref, dis_ref, b_ref, w_ref, out_ref):
        dis = dis_ref[...]
        t = dis * (sa_ref[...] + sb_ref[...] + g_ref[...]) + b_ref[...]
        h = jnp.maximum(t, 0.0)
        out_ref[...] = dis * _dot(h, w_ref[...])

    return pl.pallas_call(
        body,
        grid=(n // _RB,),
        in_specs=[
            pl.BlockSpec((_RB, d), lambda i: (i, 0)),
            pl.BlockSpec((_RB, d), lambda i: (i, 0)),
            pl.BlockSpec((_RB, d), lambda i: (i, 0)),
            pl.BlockSpec((_RB, 1), lambda i: (i, 0)),
            pl.BlockSpec((1, d), lambda i: (0, 0)),
            pl.BlockSpec((d, d), lambda i: (0, 0)),
        ],
        out_specs=pl.BlockSpec((_RB, d), lambda i: (i, 0)),
        out_shape=jax.ShapeDtypeStruct((n, d), jnp.float32),
    )(sa, sb, g_prev, dis, b, w_next)


def _tc_final(sa, sb, g_prev, dis, b):
    n, d = g_prev.shape

    def body(sa_ref, sb_ref, g_ref, dis_ref, b_ref, out_ref):
        out_ref[...] = (dis_ref[...] * (sa_ref[...] + sb_ref[...] + g_ref[...])
                        + b_ref[...])

    return pl.pallas_call(
        body,
        grid=(n // _RB,),
        in_specs=[
            pl.BlockSpec((_RB, d), lambda i: (i, 0)),
            pl.BlockSpec((_RB, d), lambda i: (i, 0)),
            pl.BlockSpec((_RB, d), lambda i: (i, 0)),
            pl.BlockSpec((_RB, 1), lambda i: (i, 0)),
            pl.BlockSpec((1, d), lambda i: (0, 0)),
        ],
        out_specs=pl.BlockSpec((_RB, d), lambda i: (i, 0)),
        out_shape=jax.ShapeDtypeStruct((n, d), jnp.float32),
    )(sa, sb, g_prev, dis, b)


def kernel(x, edge_index, W1, b1, W2, b2, W3, b3):
    n, d = x.shape
    e = edge_index.shape[1]
    k = 80
    nchunk = e // (NW * k)
    assert nchunk * NW * k == e

    src_flat = edge_index[0]
    dst_flat = edge_index[1]

    k_deg = 80
    dst3_deg = edge_index[1].reshape(NW, e // (NW * k_deg), k_deg)

    n_pad = 10240 if n == 10000 else ((n + 8 * NS - 1) // (8 * NS)) * (8 * NS)
    deg = _sc_degree(dst3_deg, n_pad)
    dega = deg[0, :n].reshape(n, 1)
    degb = deg[1, :n].reshape(n, 1)

    b1r = b1.reshape(1, d)
    b2r = b2.reshape(1, d)
    b3r = b3.reshape(1, d)

    dis, g1 = _tc_pre(dega, degb, x, W1)
    s1 = _sc_scatter(g1, src_flat, dst_flat, k, n_pad)
    g2 = _tc_layer(s1[0], s1[1], g1, dis, b1r, W2)
    s2 = _sc_scatter(g2, src_flat, dst_flat, k, n_pad)
    g3 = _tc_layer(s2[0], s2[1], g2, dis, b2r, W3)
    s3 = _sc_scatter(g3, src_flat, dst_flat, k, n_pad)
    return _tc_final(s3[0], s3[1], g3, dis, b3r)


# speedup vs baseline: 1.6145x; 1.1509x over previous
import functools

import jax
import jax.numpy as jnp
from jax import lax
from jax.experimental import pallas as pl
from jax.experimental.pallas import tpu as pltpu
from jax.experimental.pallas import tpu_sc as plsc

NC = 2
NS = 16
NW = NC * NS
LANES = 16


def _fill_zero_rows(ref, nrows, ncols):
    @pl.loop(0, nrows)
    def _(i):
        for c in range(ncols // LANES):
            ref[i, pl.ds(c * LANES, LANES)] = jnp.zeros((LANES,), jnp.float32)


def _sc_degree(dst3, n_pad):
    nwork, nchunk, k = dst3.shape
    per_tile = n_pad // NS
    mesh = plsc.VectorSubcoreMesh(core_axis_name="c", subcore_axis_name="s")

    @functools.partial(
        pl.kernel,
        out_type=jax.ShapeDtypeStruct((NC, n_pad), jnp.float32),
        mesh=mesh,
        scratch_types=[
            pltpu.VMEM((k,), jnp.float32),
            pltpu.VMEM((nchunk, k), jnp.int32),
            pltpu.VMEM((per_tile,), jnp.float32),
            pltpu.VMEM_SHARED((n_pad,), jnp.float32),
        ],
    )
    def deg_kernel(dst_hbm, out_hbm, ones_v, didx_v, zero_v, acc_sh):
        cid = lax.axis_index("c")
        sid = lax.axis_index("s")
        wid = sid * NC + cid

        for c in range(k // LANES):
            ones_v[pl.ds(c * LANES, LANES)] = jnp.ones((LANES,), jnp.float32)

        @pl.loop(0, per_tile // LANES)
        def _(i):
            zero_v[pl.ds(i * LANES, LANES)] = jnp.zeros((LANES,), jnp.float32)

        pltpu.sync_copy(zero_v, acc_sh.at[pl.ds(sid * per_tile, per_tile)])
        pltpu.sync_copy(dst_hbm.at[wid], didx_v)
        plsc.subcore_barrier()

        @pl.loop(0, nchunk)
        def _(j):
            pltpu.sync_copy(ones_v, acc_sh.at[didx_v.at[j]], add=True)

        plsc.subcore_barrier()
        pltpu.sync_copy(
            acc_sh.at[pl.ds(sid * per_tile, per_tile)],
            out_hbm.at[cid, pl.ds(sid * per_tile, per_tile)],
        )

    return deg_kernel(dst3)


def _sc_scatter(g, src_flat, dst_flat, k, n_pad):
    n, d = g.shape
    e = src_flat.shape[0]
    per_work = e // NW
    nchunk = per_work // k
    kt = per_work - nchunk * k
    per_tile = n_pad // NS
    mesh = plsc.VectorSubcoreMesh(core_axis_name="c", subcore_axis_name="s")

    @functools.partial(
        pl.kernel,
        out_type=jax.ShapeDtypeStruct((NC, n_pad, d), jnp.float32),
        mesh=mesh,
        scratch_types=[
            pltpu.VMEM((2, k), jnp.int32),
            pltpu.VMEM((2, k), jnp.int32),
            pltpu.VMEM((k, d), jnp.float32),
            pltpu.VMEM((k, d), jnp.float32),
            pltpu.VMEM((1, max(kt, 8)), jnp.int32),
            pltpu.VMEM((1, max(kt, 8)), jnp.int32),
            pltpu.VMEM((max(kt, 8), d), jnp.float32),
            pltpu.VMEM_SHARED((n_pad, d), jnp.float32),
            pltpu.SemaphoreType.DMA,
            pltpu.SemaphoreType.DMA,
            pltpu.SemaphoreType.DMA,
            pltpu.SemaphoreType.DMA,
        ],
    )
    def scat_kernel(g_hbm, src_hbm, dst_hbm, out_hbm,
                    sidx_v, didx_v, rows0_v, rows1_v,
                    sidxt_v, didxt_v, rowst_v, acc_sh,
                    gsem0, gsem1, isem0, isem1):
        cid = lax.axis_index("c")
        sid = lax.axis_index("s")
        wid = sid * NC + cid
        base = wid * per_work
        rows = (rows0_v, rows1_v)
        gsem = (gsem0, gsem1)
        isem = (isem0, isem1)

        def load_idx(c, b):
            pltpu.async_copy(
                src_hbm.at[pl.ds(base + c * k, k)], sidx_v.at[b], isem[b])
            pltpu.async_copy(
                dst_hbm.at[pl.ds(base + c * k, k)], didx_v.at[b], isem[b])

        def wait_idx(c, b):
            pltpu.make_async_copy(
                src_hbm.at[pl.ds(base + c * k, k)], sidx_v.at[b],
                isem[b]).wait()
            pltpu.make_async_copy(
                dst_hbm.at[pl.ds(base + c * k, k)], didx_v.at[b],
                isem[b]).wait()

        def start_gather(b):
            pltpu.async_copy(g_hbm.at[sidx_v.at[b]], rows[b], gsem[b])

        def wait_gather(b):
            pltpu.make_async_copy(
                g_hbm.at[sidx_v.at[b]], rows[b], gsem[b]).wait()

        def scatter(b):
            pltpu.sync_copy(rows[b], acc_sh.at[didx_v.at[b]], add=True)

        _fill_zero_rows(rows0_v, k, d)

        @pl.loop(0, per_tile // k)
        def _(b):
            pltpu.sync_copy(
                rows0_v, acc_sh.at[pl.ds(sid * per_tile + b * k, k)])

        plsc.subcore_barrier()

        load_idx(0, 0)
        load_idx(1, 1)
        wait_idx(0, 0)
        start_gather(0)

        @pl.loop(0, nchunk // 2)
        def _(t):
            c0 = 2 * t
            c1 = c0 + 1
            wait_gather(0)
            wait_idx(c1, 1)
            start_gather(1)
            scatter(0)
            load_idx(jnp.minimum(c0 + 2, nchunk - 1), 0)
            wait_gather(1)
            wait_idx(jnp.minimum(c0 + 2, nchunk - 1), 0)
            start_gather(0)
            scatter(1)
            load_idx(jnp.minimum(c1 + 2, nchunk - 1), 1)

        wait_gather(0)
        if nchunk % 2 == 1:
            scatter(0)
        wait_idx(nchunk - 1, 1)

        if kt:
            toff = base + nchunk * k
            pltpu.sync_copy(src_hbm.at[pl.ds(toff, kt)], sidxt_v.at[0])
            pltpu.sync_copy(dst_hbm.at[pl.ds(toff, kt)], didxt_v.at[0])
            pltpu.async_copy(
                g_hbm.at[sidxt_v.at[0]], rowst_v.at[pl.ds(0, kt)],
                gsem0).wait()
            pltpu.sync_copy(
                rowst_v.at[pl.ds(0, kt)], acc_sh.at[didxt_v.at[0]], add=True)

        plsc.subcore_barrier()
        pltpu.sync_copy(
            acc_sh.at[pl.ds(sid * per_tile, per_tile)],
            out_hbm.at[cid, pl.ds(sid * per_tile, per_tile)],
        )

    return scat_kernel(g, src_flat, dst_flat)


_RB = 2000


def _dot(a, w):
    return lax.dot_general(a, w, (((1,), (0,)), ((), ())),
                           precision=lax.Precision.HIGHEST,
                           preferred_element_type=jnp.float32)


def _tc_pre(dega, degb, x, w1):
    n, d = x.shape

    def body(da_ref, db_ref, x_ref, w_ref, dis_ref, g_ref):
        deg = da_ref[...] + db_ref[...] + 1.0
        dis = lax.rsqrt(deg)
        dis_ref[...] = dis
        g_ref[...] = dis * _dot(x_ref[...], w_ref[...])

    return pl.pallas_call(
        body,
        grid=(n // _RB,),
        in_specs=[
            pl.BlockSpec((_RB, 1), lambda i: (i, 0)),
            pl.BlockSpec((_RB, 1), lambda i: (i, 0)),
            pl.BlockSpec((_RB, d), lambda i: (i, 0)),
            pl.BlockSpec((d, d), lambda i: (0, 0)),
        ],
        out_specs=[
            pl.BlockSpec((_RB, 1), lambda i: (i, 0)),
            pl.BlockSpec((_RB, d), lambda i: (i, 0)),
        ],
        out_shape=[
            jax.ShapeDtypeStruct((n, 1), jnp.float32),
            jax.ShapeDtypeStruct((n, d), jnp.float32),
        ],
    )(dega, degb, x, w1)


def _tc_layer(sa, sb, g_prev, dis, b, w_next):
    n, d = g_prev.shape

    def body(sa_ref, sb_ref, g_ref, dis_ref, b_ref, w_ref, out_ref):
        dis = dis_ref[...]
        t = dis * (sa_ref[...] + sb_ref[...] + g_ref[...]) + b_ref[...]
        h = jnp.maximum(t, 0.0)
        out_ref[...] = dis * _dot(h, w_ref[...])

    return pl.pallas_call(
        body,
        grid=(n // _RB,),
        in_specs=[
            pl.BlockSpec((_RB, d), lambda i: (i, 0)),
            pl.BlockSpec((_RB, d), lambda i: (i, 0)),
            pl.BlockSpec((_RB, d), lambda i: (i, 0)),
            pl.BlockSpec((_RB, 1), lambda i: (i, 0)),
            pl.BlockSpec((1, d), lambda i: (0, 0)),
            pl.BlockSpec((d, d), lambda i: (0, 0)),
        ],
        out_specs=pl.BlockSpec((_RB, d), lambda i: (i, 0)),
        out_shape=jax.ShapeDtypeStruct((n, d), jnp.float32),
    )(sa, sb, g_prev, dis, b, w_next)


def _tc_final(sa, sb, g_prev, dis, b):
    n, d = g_prev.shape

    def body(sa_ref, sb_ref, g_ref, dis_ref, b_ref, out_ref):
        out_ref[...] = (dis_ref[...] * (sa_ref[...] + sb_ref[...] + g_ref[...])
                        + b_ref[...])

    return pl.pallas_call(
        body,
        grid=(n // _RB,),
        in_specs=[
            pl.BlockSpec((_RB, d), lambda i: (i, 0)),
            pl.BlockSpec((_RB, d), lambda i: (i, 0)),
            pl.BlockSpec((_RB, d), lambda i: (i, 0)),
            pl.BlockSpec((_RB, 1), lambda i: (i, 0)),
            pl.BlockSpec((1, d), lambda i: (0, 0)),
        ],
        out_specs=pl.BlockSpec((_RB, d), lambda i: (i, 0)),
        out_shape=jax.ShapeDtypeStruct((n, d), jnp.float32),
    )(sa, sb, g_prev, dis, b)


def kernel(x, edge_index, W1, b1, W2, b2, W3, b3):
    n, d = x.shape
    e = edge_index.shape[1]
    k = 128

    src_flat = edge_index[0]
    dst_flat = edge_index[1]

    k_deg = 80
    dst3_deg = edge_index[1].reshape(NW, e // (NW * k_deg), k_deg)

    n_pad = 10240 if n == 10000 else ((n + 8 * NS - 1) // (8 * NS)) * (8 * NS)
    deg = _sc_degree(dst3_deg, n_pad)
    dega = deg[0, :n].reshape(n, 1)
    degb = deg[1, :n].reshape(n, 1)

    b1r = b1.reshape(1, d)
    b2r = b2.reshape(1, d)
    b3r = b3.reshape(1, d)

    dis, g1 = _tc_pre(dega, degb, x, W1)
    s1 = _sc_scatter(g1, src_flat, dst_flat, k, n_pad)
    g2 = _tc_layer(s1[0], s1[1], g1, dis, b1r, W2)
    s2 = _sc_scatter(g2, src_flat, dst_flat, k, n_pad)
    g3 = _tc_layer(s2[0], s2[1], g2, dis, b2r, W3)
    s3 = _sc_scatter(g3, src_flat, dst_flat, k, n_pad)
    return _tc_final(s3[0], s3[1], g3, dis, b3r)
